# baseline (speedup 1.0000x reference)
import functools
import numpy as np
import jax
import jax.numpy as jnp
from jax import lax
from jax.experimental import pallas as pl
from jax.experimental.pallas import tpu as pltpu
from jax.experimental.pallas import tpu_sc as plsc

D_MODEL = 256
N_HEADS = 8
DH = D_MODEL // N_HEADS
N_LEVELS = 4
N_POINTS = (6, 4, 4, 2)
TP = 16
SPATIAL = ((64, 64), (32, 32), (16, 16), (8, 8))
SEQ = 5440
B = 4
Q = 900
BQ = B * Q
BQP = 3840
NTAB = B * SEQ * N_HEADS

NC = 2
NS = 16
NW = NC * NS
RPW = BQP // NW
NBLK = RPW // 8
ROWS_SUB = 2 * 4 * 128

_LVL = np.repeat(np.arange(N_LEVELS), N_POINTS)
_WF = np.tile(np.array([SPATIAL[l][1] for l in _LVL], np.float32), N_HEADS)
_HF = np.tile(np.array([SPATIAL[l][0] for l in _LVL], np.float32), N_HEADS)
_ST = np.tile(np.array(
    [int(np.sum([h * w for h, w in SPATIAL[:l]])) for l in _LVL],
    np.int32), N_HEADS)
_BD = (np.arange(128)[:, None] // TP == np.arange(128)[None, :] // TP
       ).astype(np.float32)


def _mm_kernel(x_ref, w_ref, b_ref, o_ref):
    o_ref[...] = (
        jnp.dot(x_ref[...], w_ref[...], preferred_element_type=jnp.float32)
        + b_ref[...])


def _val_kernel(x_ref, w_ref, b_ref, o_ref):
    acc = (jnp.dot(x_ref[...], w_ref[...], preferred_element_type=jnp.float32)
           + b_ref[...])
    u = lax.bitcast_convert_type(
        acc.astype(jnp.bfloat16), jnp.uint16).astype(jnp.int32)
    words = []
    for h in range(N_HEADS):
        lo = u[:, h * DH:h * DH + TP]
        hi = u[:, h * DH + TP:h * DH + DH]
        words.append(lo | lax.shift_left(hi, 16))
    o_ref[...] = jnp.concatenate(words, axis=1)


def _val_matmul(x, w, b, bm):
    m, k = x.shape
    n = w.shape[1]
    return pl.pallas_call(
        _val_kernel,
        grid=(m // bm,),
        in_specs=[
            pl.BlockSpec((bm, k), lambda i: (i, 0)),
            pl.BlockSpec((k, n), lambda i: (0, 0)),
            pl.BlockSpec((1, n), lambda i: (0, 0)),
        ],
        out_specs=pl.BlockSpec((bm, n // 2), lambda i: (i, 0)),
        out_shape=jax.ShapeDtypeStruct((m, n // 2), jnp.int32),
    )(x, w, b.reshape(1, n))


def _matmul(x, w, b, bm):
    m, k = x.shape
    n = w.shape[1]
    return pl.pallas_call(
        _mm_kernel,
        grid=(m // bm,),
        in_specs=[
            pl.BlockSpec((bm, k), lambda i: (i, 0)),
            pl.BlockSpec((k, n), lambda i: (0, 0)),
            pl.BlockSpec((1, n), lambda i: (0, 0)),
        ],
        out_specs=pl.BlockSpec((bm, n), lambda i: (i, 0)),
        out_shape=jax.ShapeDtypeStruct((m, n), jnp.float32),
    )(x, w, b.reshape(1, n))


def _samp_kernel(q_ref, wc_ref, bc_ref, rp_ref, ex_ref, ey_ref,
                 bd_ref, wf_ref, hf_ref, st_ref,
                 i0_ref, i1_ref, i2_ref, i3_ref,
                 w0_ref, w1_ref, w2_ref, w3_ref, *, blk):
    oa = (jnp.dot(q_ref[...], wc_ref[...],
                  preferred_element_type=jnp.float32) + bc_ref[...])
    rp = rp_ref[...]
    rx = jnp.dot(rp, ex_ref[...], preferred_element_type=jnp.float32,
                 precision=lax.Precision.HIGHEST)
    ry = jnp.dot(rp, ey_ref[...], preferred_element_type=jnp.float32,
                 precision=lax.Precision.HIGHEST)
    offx = oa[:, 0:128]
    offy = oa[:, 128:256]
    lg = oa[:, 256:384]
    mx = jnp.max(lg, axis=-1, keepdims=True)
    e = jnp.exp(lg - mx)
    esum = jnp.dot(e, bd_ref[...], preferred_element_type=jnp.float32)
    attn = e / esum

    wf = wf_ref[...]
    hf = hf_ref[...]
    st = st_ref[...]
    wi = wf.astype(jnp.int32)

    locx = rx + offx / wf
    locy = ry + offy / hf
    xf = locx * wf - 0.5
    yf = locy * hf - 0.5
    x0 = jnp.floor(xf)
    y0 = jnp.floor(yf)
    wx1 = xf - x0
    wx0 = 1.0 - wx1
    wy1 = yf - y0
    wy0 = 1.0 - wy1

    row0 = pl.program_id(0) * blk + lax.broadcasted_iota(
        jnp.int32, (blk, 1), 0)
    b = jnp.minimum(row0 // Q, B - 1)
    h = lax.broadcasted_iota(jnp.int32, (1, 128), 1) // TP
    base = b * (SEQ * N_HEADS) + h

    def corner(xc, yc, wx, wy):
        vx = (xc >= 0.0) & (xc < wf)
        vy = (yc >= 0.0) & (yc < hf)
        ixc = jnp.clip(xc, 0.0, wf - 1.0).astype(jnp.int32)
        iyc = jnp.clip(yc, 0.0, hf - 1.0).astype(jnp.int32)
        s = st + iyc * wi + ixc
        idx = jnp.minimum(base + s * N_HEADS, NTAB - 1)
        w = attn * wx * wy * (vx & vy).astype(jnp.float32)
        return idx, w

    x1 = x0 + 1.0
    y1 = y0 + 1.0
    i0, w0 = corner(x0, y0, wx0, wy0)
    i1, w1 = corner(x1, y0, wx1, wy0)
    i2, w2 = corner(x0, y1, wx0, wy1)
    i3, w3 = corner(x1, y1, wx1, wy1)
    i0_ref[...] = i0
    i1_ref[...] = i1
    i2_ref[...] = i2
    i3_ref[...] = i3
    w0_ref[...] = w0
    w1_ref[...] = w1
    w2_ref[...] = w2
    w3_ref[...] = w3


def _sampling(q2, wc, bc, rp, blk=768):
    spec_q = pl.BlockSpec((blk, D_MODEL), lambda i: (i, 0))
    spec_wc = pl.BlockSpec((D_MODEL, 384), lambda i: (0, 0))
    spec_bc = pl.BlockSpec((1, 384), lambda i: (0, 0))
    spec_rp = pl.BlockSpec((blk, 128), lambda i: (i, 0))
    spec_e = pl.BlockSpec((128, 128), lambda i: (0, 0))
    spec2 = pl.BlockSpec((blk, 128), lambda i: (i, 0))
    spec_bd = pl.BlockSpec((128, 128), lambda i: (0, 0))
    spec_c = pl.BlockSpec((1, 128), lambda i: (0, 0))
    shp_i = jax.ShapeDtypeStruct((BQP, 128), jnp.int32)
    shp_f = jax.ShapeDtypeStruct((BQP, 128), jnp.float32)
    lvl = np.tile(_LVL, N_HEADS)
    ex = (np.arange(128)[:, None] == 2 * lvl[None, :]).astype(np.float32)
    ey = (np.arange(128)[:, None] == 2 * lvl[None, :] + 1).astype(np.float32)
    return pl.pallas_call(
        functools.partial(_samp_kernel, blk=blk),
        grid=(BQP // blk,),
        in_specs=[spec_q, spec_wc, spec_bc, spec_rp, spec_e, spec_e,
                  spec_bd] + [spec_c] * 3,
        out_specs=[spec2] * 8,
        out_shape=[shp_i] * 4 + [shp_f] * 4,
    )(q2, wc, bc.reshape(1, 384), rp, jnp.asarray(ex), jnp.asarray(ey),
      jnp.asarray(_BD), jnp.asarray(_WF).reshape(1, 128),
      jnp.asarray(_HF).reshape(1, 128), jnp.asarray(_ST).reshape(1, 128))


def _sc_body(val_hbm, i0_hbm, i1_hbm, i2_hbm, i3_hbm,
             w0_hbm, w1_hbm, w2_hbm, w3_hbm, out_hbm,
             idx_v, w_v, rows_v, out_v, sem0, sem1, sem_st):
    wid = lax.axis_index("s") * NC + lax.axis_index("c")
    sems = (sem0, sem1)
    idx_hbms = (i0_hbm, i1_hbm, i2_hbm, i3_hbm)
    w_hbms = (w0_hbm, w1_hbm, w2_hbm, w3_hbm)

    def stage(t, slot):
        base8 = wid * RPW + t * 8
        for c in range(4):
            pltpu.async_copy(
                idx_hbms[c].at[pl.ds(base8, 8)], idx_v.at[slot, c], sem_st)
            pltpu.async_copy(
                w_hbms[c].at[pl.ds(base8, 8)], w_v.at[slot, c], sem_st)

    def wait_stage():
        for c in range(4):
            pltpu.make_async_copy(
                i0_hbm.at[pl.ds(0, 8)], idx_v.at[0, c], sem_st).wait()
            pltpu.make_async_copy(
                w0_hbm.at[pl.ds(0, 8)], w_v.at[0, c], sem_st).wait()

    def fire(slot, s, rbuf):
        for c in range(4):
            for lbq in range(2):
                pltpu.async_copy(
                    val_hbm.at[idx_v.at[slot, c, 2 * s + lbq]],
                    rows_v.at[rbuf, pl.ds((c * 2 + lbq) * 128, 128)],
                    sems[rbuf])

    def wait_rows(rbuf):
        pltpu.make_async_copy(
            val_hbm.at[pl.ds(0, ROWS_SUB)], rows_v.at[rbuf],
            sems[rbuf]).wait()

    def compute(slot, s, rbuf):
        for lbq in range(2):
            r8 = 2 * s + lbq

            def hbody(h, _):
                parts = []
                for c in range(4):
                    wv = w_v[slot, c, r8, pl.ds(h * TP, TP)]

                    def pbody(p, a, c=c, wv=wv):
                        a0, a1 = a
                        wsp = wv[jnp.full((16,), p, jnp.int32)]
                        row = (c * 2 + lbq) * 128 + h * TP + p
                        v = rows_v[rbuf, row, pl.ds(0, 16)]
                        ve = plsc.bitcast(
                            lax.shift_left(v, 16), jnp.float32)
                        vo = plsc.bitcast(
                            v & jnp.int32(-65536), jnp.float32)
                        return (a0 + wsp * ve, a1 + wsp * vo)

                    parts.append(lax.fori_loop(
                        0, TP, pbody,
                        (jnp.zeros((16,), jnp.float32),
                         jnp.zeros((16,), jnp.float32)), unroll=16))
                out_v[r8, pl.ds(h * DH, 16)] = (
                    (parts[0][0] + parts[1][0]) + (parts[2][0] + parts[3][0]))
                out_v[r8, pl.ds(h * DH + 16, 16)] = (
                    (parts[0][1] + parts[1][1]) + (parts[2][1] + parts[3][1]))
                return 0

            lax.fori_loop(0, N_HEADS, hbody, 0)

    stage(0, 0)
    wait_stage()
    fire(0, 0, 0)
    fire(0, 1, 1)

    def outer(t, carry):
        slot = lax.rem(t, 2)
        nslot = 1 - slot
        more = t + 1 < NBLK

        @pl.when(more)
        def _():
            stage(t + 1, nslot)

        wait_rows(0)
        compute(slot, 0, 0)
        fire(slot, 2, 0)
        wait_rows(1)
        compute(slot, 1, 1)
        fire(slot, 3, 1)

        @pl.when(more)
        def _():
            wait_stage()

        wait_rows(0)
        compute(slot, 2, 0)

        @pl.when(more)
        def _():
            fire(nslot, 0, 0)

        wait_rows(1)
        compute(slot, 3, 1)

        @pl.when(more)
        def _():
            fire(nslot, 1, 1)

        pltpu.sync_copy(out_v, out_hbm.at[pl.ds(wid * RPW + t * 8, 8)])
        return carry

    lax.fori_loop(0, NBLK, outer, 0)


def _sc_gather(val_tab, idxs, ws):
    mesh = plsc.VectorSubcoreMesh(core_axis_name="c", subcore_axis_name="s")
    f = functools.partial(
        pl.kernel,
        out_type=jax.ShapeDtypeStruct((BQP, D_MODEL), jnp.float32),
        mesh=mesh,
        compiler_params=pltpu.CompilerParams(
            needs_layout_passes=False, use_tc_tiling_on_sc=False),
        scratch_types=[
            pltpu.VMEM((2, 4, 8, 128), jnp.int32),
            pltpu.VMEM((2, 4, 8, 128), jnp.float32),
            pltpu.VMEM((2, ROWS_SUB, TP), jnp.int32),
            pltpu.VMEM((8, D_MODEL), jnp.float32),
            pltpu.SemaphoreType.DMA,
            pltpu.SemaphoreType.DMA,
            pltpu.SemaphoreType.DMA,
        ],
    )(_sc_body)
    return f(val_tab, *idxs, *ws)


def kernel(query, reference_points, input_flatten, input_spatial_shapes,
           input_level_start_index, W_value, b_value, W_off, b_off, W_attn,
           b_attn, W_out, b_out):
    x = input_flatten.reshape(B * SEQ, D_MODEL)
    val = _val_matmul(x, W_value, b_value, bm=1360)
    val_tab = val.reshape(NTAB, TP)

    wc = jnp.concatenate([W_off[:, 0::2], W_off[:, 1::2], W_attn], axis=1)
    bc = jnp.concatenate([b_off[0::2], b_off[1::2], b_attn], axis=0)
    q2 = jnp.pad(query.reshape(BQ, D_MODEL), ((0, BQP - BQ), (0, 0)))
    rp = jnp.pad(reference_points.reshape(BQ, N_LEVELS * 2),
                 ((0, BQP - BQ), (0, 128 - N_LEVELS * 2)))

    i0, i1, i2, i3, w0, w1, w2, w3 = _sampling(q2, wc, bc, rp)

    msda = _sc_gather(val_tab, (i0, i1, i2, i3), (w0, w1, w2, w3))

    out = _matmul(msda[:BQ], W_out, b_out, bm=720)
    return out.reshape(B, Q, D_MODEL)


# speedup vs baseline: 1.0095x; 1.0095x over previous
import functools
import numpy as np
import jax
import jax.numpy as jnp
from jax import lax
from jax.experimental import pallas as pl
from jax.experimental.pallas import tpu as pltpu
from jax.experimental.pallas import tpu_sc as plsc

D_MODEL = 256
N_HEADS = 8
DH = D_MODEL // N_HEADS
N_LEVELS = 4
N_POINTS = (6, 4, 4, 2)
TP = 16
SPATIAL = ((64, 64), (32, 32), (16, 16), (8, 8))
SEQ = 5440
B = 4
Q = 900
BQ = B * Q
BQP = 3840
NTAB = B * SEQ * N_HEADS

NC = 2
NS = 16
NW = NC * NS
RPW = BQP // NW
NBLK = RPW // 8
SUBQ = 4
ROWS_SUB = SUBQ * 4 * 128

_LVL = np.repeat(np.arange(N_LEVELS), N_POINTS)
_WF = np.tile(np.array([SPATIAL[l][1] for l in _LVL], np.float32), N_HEADS)
_HF = np.tile(np.array([SPATIAL[l][0] for l in _LVL], np.float32), N_HEADS)
_ST = np.tile(np.array(
    [int(np.sum([h * w for h, w in SPATIAL[:l]])) for l in _LVL],
    np.int32), N_HEADS)
_BD = (np.arange(128)[:, None] // TP == np.arange(128)[None, :] // TP
       ).astype(np.float32)


def _mm_kernel(x_ref, w_ref, b_ref, o_ref):
    o_ref[...] = (
        jnp.dot(x_ref[...], w_ref[...], preferred_element_type=jnp.float32)
        + b_ref[...])


def _val_kernel(x_ref, w_ref, b_ref, o_ref):
    acc = (jnp.dot(x_ref[...], w_ref[...], preferred_element_type=jnp.float32)
           + b_ref[...])
    u = lax.bitcast_convert_type(
        acc.astype(jnp.bfloat16), jnp.uint16).astype(jnp.int32)
    words = []
    for h in range(N_HEADS):
        lo = u[:, h * DH:h * DH + TP]
        hi = u[:, h * DH + TP:h * DH + DH]
        words.append(lo | lax.shift_left(hi, 16))
    o_ref[...] = jnp.concatenate(words, axis=1)


def _val_matmul(x, w, b, bm):
    m, k = x.shape
    n = w.shape[1]
    return pl.pallas_call(
        _val_kernel,
        grid=(m // bm,),
        in_specs=[
            pl.BlockSpec((bm, k), lambda i: (i, 0)),
            pl.BlockSpec((k, n), lambda i: (0, 0)),
            pl.BlockSpec((1, n), lambda i: (0, 0)),
        ],
        out_specs=pl.BlockSpec((bm, n // 2), lambda i: (i, 0)),
        out_shape=jax.ShapeDtypeStruct((m, n // 2), jnp.int32),
    )(x, w, b.reshape(1, n))


def _matmul(x, w, b, bm):
    m, k = x.shape
    n = w.shape[1]
    return pl.pallas_call(
        _mm_kernel,
        grid=(m // bm,),
        in_specs=[
            pl.BlockSpec((bm, k), lambda i: (i, 0)),
            pl.BlockSpec((k, n), lambda i: (0, 0)),
            pl.BlockSpec((1, n), lambda i: (0, 0)),
        ],
        out_specs=pl.BlockSpec((bm, n), lambda i: (i, 0)),
        out_shape=jax.ShapeDtypeStruct((m, n), jnp.float32),
    )(x, w, b.reshape(1, n))


def _samp_kernel(q_ref, wc_ref, bc_ref, rp_ref, ex_ref, ey_ref,
                 bd_ref, wf_ref, hf_ref, st_ref,
                 i0_ref, i1_ref, i2_ref, i3_ref,
                 w0_ref, w1_ref, w2_ref, w3_ref, *, blk):
    oa = (jnp.dot(q_ref[...], wc_ref[...],
                  preferred_element_type=jnp.float32) + bc_ref[...])
    rp = rp_ref[...]
    rx = jnp.dot(rp, ex_ref[...], preferred_element_type=jnp.float32,
                 precision=lax.Precision.HIGHEST)
    ry = jnp.dot(rp, ey_ref[...], preferred_element_type=jnp.float32,
                 precision=lax.Precision.HIGHEST)
    offx = oa[:, 0:128]
    offy = oa[:, 128:256]
    lg = oa[:, 256:384]
    mx = jnp.max(lg, axis=-1, keepdims=True)
    e = jnp.exp(lg - mx)
    esum = jnp.dot(e, bd_ref[...], preferred_element_type=jnp.float32)
    attn = e / esum

    wf = wf_ref[...]
    hf = hf_ref[...]
    st = st_ref[...]
    wi = wf.astype(jnp.int32)

    locx = rx + offx / wf
    locy = ry + offy / hf
    xf = locx * wf - 0.5
    yf = locy * hf - 0.5
    x0 = jnp.floor(xf)
    y0 = jnp.floor(yf)
    wx1 = xf - x0
    wx0 = 1.0 - wx1
    wy1 = yf - y0
    wy0 = 1.0 - wy1

    row0 = pl.program_id(0) * blk + lax.broadcasted_iota(
        jnp.int32, (blk, 1), 0)
    b = jnp.minimum(row0 // Q, B - 1)
    h = lax.broadcasted_iota(jnp.int32, (1, 128), 1) // TP
    base = b * (SEQ * N_HEADS) + h

    def corner(xc, yc, wx, wy):
        vx = (xc >= 0.0) & (xc < wf)
        vy = (yc >= 0.0) & (yc < hf)
        ixc = jnp.clip(xc, 0.0, wf - 1.0).astype(jnp.int32)
        iyc = jnp.clip(yc, 0.0, hf - 1.0).astype(jnp.int32)
        s = st + iyc * wi + ixc
        idx = jnp.minimum(base + s * N_HEADS, NTAB - 1)
        w = attn * wx * wy * (vx & vy).astype(jnp.float32)
        return idx, w

    x1 = x0 + 1.0
    y1 = y0 + 1.0
    i0, w0 = corner(x0, y0, wx0, wy0)
    i1, w1 = corner(x1, y0, wx1, wy0)
    i2, w2 = corner(x0, y1, wx0, wy1)
    i3, w3 = corner(x1, y1, wx1, wy1)
    i0_ref[...] = i0
    i1_ref[...] = i1
    i2_ref[...] = i2
    i3_ref[...] = i3
    w0_ref[...] = w0
    w1_ref[...] = w1
    w2_ref[...] = w2
    w3_ref[...] = w3


def _sampling(q2, wc, bc, rp, blk=768):
    spec_q = pl.BlockSpec((blk, D_MODEL), lambda i: (i, 0))
    spec_wc = pl.BlockSpec((D_MODEL, 384), lambda i: (0, 0))
    spec_bc = pl.BlockSpec((1, 384), lambda i: (0, 0))
    spec_rp = pl.BlockSpec((blk, 128), lambda i: (i, 0))
    spec_e = pl.BlockSpec((128, 128), lambda i: (0, 0))
    spec2 = pl.BlockSpec((blk, 128), lambda i: (i, 0))
    spec_bd = pl.BlockSpec((128, 128), lambda i: (0, 0))
    spec_c = pl.BlockSpec((1, 128), lambda i: (0, 0))
    shp_i = jax.ShapeDtypeStruct((BQP, 128), jnp.int32)
    shp_f = jax.ShapeDtypeStruct((BQP, 128), jnp.float32)
    lvl = np.tile(_LVL, N_HEADS)
    ex = (np.arange(128)[:, None] == 2 * lvl[None, :]).astype(np.float32)
    ey = (np.arange(128)[:, None] == 2 * lvl[None, :] + 1).astype(np.float32)
    return pl.pallas_call(
        functools.partial(_samp_kernel, blk=blk),
        grid=(BQP // blk,),
        in_specs=[spec_q, spec_wc, spec_bc, spec_rp, spec_e, spec_e,
                  spec_bd] + [spec_c] * 3,
        out_specs=[spec2] * 8,
        out_shape=[shp_i] * 4 + [shp_f] * 4,
    )(q2, wc, bc.reshape(1, 384), rp, jnp.asarray(ex), jnp.asarray(ey),
      jnp.asarray(_BD), jnp.asarray(_WF).reshape(1, 128),
      jnp.asarray(_HF).reshape(1, 128), jnp.asarray(_ST).reshape(1, 128))


def _sc_body(val_hbm, i0_hbm, i1_hbm, i2_hbm, i3_hbm,
             w0_hbm, w1_hbm, w2_hbm, w3_hbm, out_hbm,
             idx_v, w_v, rows_v, out_v, sem0, sem1, sem_st):
    wid = lax.axis_index("s") * NC + lax.axis_index("c")
    sems = (sem0, sem1)
    idx_hbms = (i0_hbm, i1_hbm, i2_hbm, i3_hbm)
    w_hbms = (w0_hbm, w1_hbm, w2_hbm, w3_hbm)

    def stage(t, slot):
        base8 = wid * RPW + t * 8
        for c in range(4):
            pltpu.async_copy(
                idx_hbms[c].at[pl.ds(base8, 8)], idx_v.at[slot, c], sem_st)
            pltpu.async_copy(
                w_hbms[c].at[pl.ds(base8, 8)], w_v.at[slot, c], sem_st)

    def wait_stage():
        for c in range(4):
            pltpu.make_async_copy(
                i0_hbm.at[pl.ds(0, 8)], idx_v.at[0, c], sem_st).wait()
            pltpu.make_async_copy(
                w0_hbm.at[pl.ds(0, 8)], w_v.at[0, c], sem_st).wait()

    def fire(slot, s, rbuf):
        for c in range(4):
            for lbq in range(SUBQ):
                pltpu.async_copy(
                    val_hbm.at[idx_v.at[slot, c, SUBQ * s + lbq]],
                    rows_v.at[rbuf, pl.ds((c * SUBQ + lbq) * 128, 128)],
                    sems[rbuf])

    def wait_rows(rbuf):
        pltpu.make_async_copy(
            val_hbm.at[pl.ds(0, ROWS_SUB)], rows_v.at[rbuf],
            sems[rbuf]).wait()

    def compute(slot, s, rbuf):
        for lbq in range(SUBQ):
            r8 = SUBQ * s + lbq

            def hbody(h, _):
                parts = []
                for c in range(4):
                    wv = w_v[slot, c, r8, pl.ds(h * TP, TP)]

                    def pbody(p, a, c=c, wv=wv):
                        a0, a1 = a
                        wsp = wv[jnp.full((16,), p, jnp.int32)]
                        row = (c * SUBQ + lbq) * 128 + h * TP + p
                        v = rows_v[rbuf, row, pl.ds(0, 16)]
                        ve = plsc.bitcast(
                            lax.shift_left(v, 16), jnp.float32)
                        vo = plsc.bitcast(
                            v & jnp.int32(-65536), jnp.float32)
                        return (a0 + wsp * ve, a1 + wsp * vo)

                    parts.append(lax.fori_loop(
                        0, TP, pbody,
                        (jnp.zeros((16,), jnp.float32),
                         jnp.zeros((16,), jnp.float32)), unroll=16))
                out_v[r8, pl.ds(h * DH, 16)] = (
                    (parts[0][0] + parts[1][0]) + (parts[2][0] + parts[3][0]))
                out_v[r8, pl.ds(h * DH + 16, 16)] = (
                    (parts[0][1] + parts[1][1]) + (parts[2][1] + parts[3][1]))
                return 0

            lax.fori_loop(0, N_HEADS, hbody, 0)

    stage(0, 0)
    wait_stage()
    fire(0, 0, 0)
    fire(0, 1, 1)

    def outer(t, carry):
        slot = lax.rem(t, 2)
        nslot = 1 - slot
        more = t + 1 < NBLK

        @pl.when(more)
        def _():
            stage(t + 1, nslot)

        wait_rows(0)
        compute(slot, 0, 0)

        @pl.when(more)
        def _():
            wait_stage()
            fire(nslot, 0, 0)

        wait_rows(1)
        compute(slot, 1, 1)

        @pl.when(more)
        def _():
            fire(nslot, 1, 1)

        pltpu.sync_copy(out_v, out_hbm.at[pl.ds(wid * RPW + t * 8, 8)])
        return carry

    lax.fori_loop(0, NBLK, outer, 0)


def _sc_gather(val_tab, idxs, ws):
    mesh = plsc.VectorSubcoreMesh(core_axis_name="c", subcore_axis_name="s")
    f = functools.partial(
        pl.kernel,
        out_type=jax.ShapeDtypeStruct((BQP, D_MODEL), jnp.float32),
        mesh=mesh,
        compiler_params=pltpu.CompilerParams(
            needs_layout_passes=False, use_tc_tiling_on_sc=False),
        scratch_types=[
            pltpu.VMEM((2, 4, 8, 128), jnp.int32),
            pltpu.VMEM((2, 4, 8, 128), jnp.float32),
            pltpu.VMEM((2, ROWS_SUB, TP), jnp.int32),
            pltpu.VMEM((8, D_MODEL), jnp.float32),
            pltpu.SemaphoreType.DMA,
            pltpu.SemaphoreType.DMA,
            pltpu.SemaphoreType.DMA,
        ],
    )(_sc_body)
    return f(val_tab, *idxs, *ws)


def kernel(query, reference_points, input_flatten, input_spatial_shapes,
           input_level_start_index, W_value, b_value, W_off, b_off, W_attn,
           b_attn, W_out, b_out):
    x = input_flatten.reshape(B * SEQ, D_MODEL)
    val = _val_matmul(x, W_value, b_value, bm=1360)
    val_tab = val.reshape(NTAB, TP)

    wc = jnp.concatenate([W_off[:, 0::2], W_off[:, 1::2], W_attn], axis=1)
    bc = jnp.concatenate([b_off[0::2], b_off[1::2], b_attn], axis=0)
    q2 = jnp.pad(query.reshape(BQ, D_MODEL), ((0, BQP - BQ), (0, 0)))
    rp = jnp.pad(reference_points.reshape(BQ, N_LEVELS * 2),
                 ((0, BQP - BQ), (0, 128 - N_LEVELS * 2)))

    i0, i1, i2, i3, w0, w1, w2, w3 = _sampling(q2, wc, bc, rp)

    msda = _sc_gather(val_tab, (i0, i1, i2, i3), (w0, w1, w2, w3))

    out = _matmul(msda[:BQ], W_out, b_out, bm=720)
    return out.reshape(B, Q, D_MODEL)
